# baseline (speedup 1.0000x reference)
import numpy as np
import jax
import jax.numpy as jnp
from jax.experimental import pallas as pl


def _pe_table(seq_len, d_model):
    position = np.arange(seq_len, dtype=np.float32)[:, None]
    div_term = np.exp(
        np.arange(0, d_model, 2, dtype=np.float32) * (-np.log(10000.0) / d_model)
    )
    pe = np.zeros((seq_len, d_model), dtype=np.float32)
    pe[:, 0::2] = np.sin(position * div_term)
    pe[:, 1::2] = np.cos(position * div_term)
    return pe


def _revert_body(tb_ref, idx_ref, pe_ref, mod_ref, mask_ref, out_ref):
    d = pe_ref.shape[1]
    ts = pe_ref.shape[0]
    pe_b = pe_ref[...]
    rows = [tb_ref[:, m * d : (m + 1) * d] for m in range(5)]
    mask_b = jnp.broadcast_to(mask_ref[0:1, :], (ts, d))
    out_ref[:, 0:d] = rows[0] + pe_b + mod_ref[0:1, :]
    for j in range(1, 9):
        ij = idx_ref[:, j - 1 : j]
        v = mask_b
        for m in range(4):
            v = jnp.where(ij == m, rows[1 + m], v)
        out_ref[:, j * d : (j + 1) * d] = v + pe_b + mod_ref[j : j + 1, :]


def kernel(temporal_block, mod_emb_weight, mask_token, temporal_revert_idx,
           temporal_masked_idx):
    b, s, m1, d = temporal_block.shape
    r = temporal_revert_idx.shape[-1]
    n = b * s

    tb = temporal_block.reshape(n, m1 * d)
    idx = temporal_revert_idx.reshape(n, r).astype(jnp.int32)
    pe = jnp.asarray(_pe_table(s, d))
    mod9 = mod_emb_weight[: r + 1]

    ts = 256
    grid = (n // ts,)
    out = pl.pallas_call(
        _revert_body,
        grid=grid,
        in_specs=[
            pl.BlockSpec((ts, m1 * d), lambda i: (i, 0)),
            pl.BlockSpec((ts, r), lambda i: (i, 0)),
            pl.BlockSpec((ts, d), lambda i: (i % (s // ts), 0)),
            pl.BlockSpec((r + 1, d), lambda i: (0, 0)),
            pl.BlockSpec((1, d), lambda i: (0, 0)),
        ],
        out_specs=pl.BlockSpec((ts, (r + 1) * d), lambda i: (i, 0)),
        out_shape=jax.ShapeDtypeStruct((n, (r + 1) * d), jnp.float32),
    )(tb, idx, pe, mod9, mask_token)
    return out.reshape(b, s, r + 1, d)


# speedup vs baseline: 1.6588x; 1.6588x over previous
import numpy as np
import jax
import jax.numpy as jnp
from jax.experimental import pallas as pl


def _pe_table(seq_len, d_model):
    position = np.arange(seq_len, dtype=np.float32)[:, None]
    div_term = np.exp(
        np.arange(0, d_model, 2, dtype=np.float32) * (-np.log(10000.0) / d_model)
    )
    pe = np.zeros((seq_len, d_model), dtype=np.float32)
    pe[:, 0::2] = np.sin(position * div_term)
    pe[:, 1::2] = np.cos(position * div_term)
    return pe


def _revert_body(tb_ref, idx_ref, pe_ref, mod_ref, mask_ref, out_ref):
    d = pe_ref.shape[1]
    ts = pe_ref.shape[0]
    pe_b = pe_ref[...]
    rows = [tb_ref[:, m, :] for m in range(5)]
    mask_b = jnp.broadcast_to(mask_ref[0:1, :], (ts, d))
    out_ref[:, 0, :] = rows[0] + pe_b + mod_ref[0:1, :]
    for j in range(1, 9):
        ij = idx_ref[:, j - 1 : j]
        v = mask_b
        for m in range(4):
            v = jnp.where(ij == m, rows[1 + m], v)
        out_ref[:, j, :] = v + pe_b + mod_ref[j : j + 1, :]


def kernel(temporal_block, mod_emb_weight, mask_token, temporal_revert_idx,
           temporal_masked_idx):
    b, s, m1, d = temporal_block.shape
    r = temporal_revert_idx.shape[-1]
    n = b * s

    tb = temporal_block.reshape(n, m1, d)
    idx = temporal_revert_idx.reshape(n, r).astype(jnp.int32)
    pe = jnp.asarray(_pe_table(s, d))
    mod9 = mod_emb_weight[: r + 1]

    ts = 256
    grid = (n // ts,)
    out = pl.pallas_call(
        _revert_body,
        grid=grid,
        in_specs=[
            pl.BlockSpec((ts, m1, d), lambda i: (i, 0, 0)),
            pl.BlockSpec((ts, r), lambda i: (i, 0)),
            pl.BlockSpec((ts, d), lambda i: (i % (s // ts), 0)),
            pl.BlockSpec((r + 1, d), lambda i: (0, 0)),
            pl.BlockSpec((1, d), lambda i: (0, 0)),
        ],
        out_specs=pl.BlockSpec((ts, r + 1, d), lambda i: (i, 0, 0)),
        out_shape=jax.ShapeDtypeStruct((n, r + 1, d), jnp.float32),
    )(tb, idx, pe, mod9, mask_token)
    return out.reshape(b, s, r + 1, d)
